# baseline (speedup 1.0000x reference)
import jax
import jax.numpy as jnp
from jax import lax
from jax.experimental import pallas as pl
from jax.experimental.pallas import tpu as pltpu
from jax.experimental.pallas import tpu_sc as plsc

NC = 2
NS = 16
NW = NC * NS
BATCH = 16384
PER_W = BATCH // NW


def _body(steps_hbm, ab_hbm, abp_hbm, a_hbm,
          out_ab, out_abp, out_a,
          idx_v, t_ab, t_abp, t_a, r_ab, r_abp, r_a, sem_t, sem_in, sem_out):
    sid = lax.axis_index("s")
    wid = sid * NC + lax.axis_index("c")
    base = wid * PER_W

    @pl.when(sid == 0)
    def _stage_ab():
        pltpu.async_copy(ab_hbm, t_ab, sem_t).wait()

    @pl.when(sid == 1)
    def _stage_abp():
        pltpu.async_copy(abp_hbm, t_abp, sem_t).wait()

    @pl.when(sid == 2)
    def _stage_a():
        pltpu.async_copy(a_hbm, t_a, sem_t).wait()

    ic = pltpu.async_copy(steps_hbm.at[pl.ds(base, PER_W)], idx_v, sem_out)
    plsc.subcore_barrier()
    ic.wait()

    g1 = pltpu.async_copy(t_ab.at[idx_v], r_ab, sem_in)
    g2 = pltpu.async_copy(t_abp.at[idx_v], r_abp, sem_in)
    g3 = pltpu.async_copy(t_a.at[idx_v], r_a, sem_in)
    g1.wait()
    s1 = pltpu.async_copy(r_ab, out_ab.at[pl.ds(base, PER_W)], sem_out)
    g2.wait()
    s2 = pltpu.async_copy(r_abp, out_abp.at[pl.ds(base, PER_W)], sem_out)
    g3.wait()
    s3 = pltpu.async_copy(r_a, out_a.at[pl.ds(base, PER_W)], sem_out)
    s1.wait()
    s2.wait()
    s3.wait()


@jax.jit
def _run(steps, ab, abp, a):
    f32 = jnp.float32
    out = jax.ShapeDtypeStruct((BATCH,), f32)
    k = pl.kernel(
        _body,
        out_type=(out, out, out),
        mesh=plsc.VectorSubcoreMesh(core_axis_name="c", subcore_axis_name="s",
                                    num_cores=NC),
        scratch_types=[
            pltpu.VMEM((PER_W,), jnp.int32),
            pltpu.VMEM_SHARED((1000,), f32),
            pltpu.VMEM_SHARED((1000,), f32),
            pltpu.VMEM_SHARED((1000,), f32),
            pltpu.VMEM((PER_W,), f32),
            pltpu.VMEM((PER_W,), f32),
            pltpu.VMEM((PER_W,), f32),
            pltpu.SemaphoreType.DMA,
            pltpu.SemaphoreType.DMA,
            pltpu.SemaphoreType.DMA,
        ],
    )
    return k(steps, ab, abp, a)


def kernel(diffusion_steps, alpha_bars, alpha_bars_prev, alphas):
    steps = diffusion_steps.astype(jnp.int32)
    ab, abp, a = _run(steps, alpha_bars, alpha_bars_prev, alphas)
    shape = (-1, 1, 1, 1)
    return (ab.reshape(shape), abp.reshape(shape), a.reshape(shape))
